# baseline (speedup 1.0000x reference)
import functools
import math

import jax
import jax.numpy as jnp
from jax import lax
from jax.experimental import pallas as pl
from jax.experimental.pallas import tpu as pltpu
from jax.experimental.pallas import tpu_sc as plsc

_E = 64
_K = 2
_H = 768
_I = 1536
_T = 2048
_C = 64
_NSLOT = _E * _C
_DUMMY = _NSLOT
_NW = 32
_SLOTS_PER_W = _NSLOT // _NW
_TOK_PER_W = _T // _NW
_CBLK = 256


def _router_body(x_ref, rw_ref, logits_ref, slot1_ref, slot2_ref,
                 tok2d_ref, kw2d_ref, ohsum_ref, cum_ref):
    x = x_ref[...]
    rw = rw_ref[...]
    logits = jnp.dot(x, rw, preferred_element_type=jnp.float32)
    logits_ref[...] = logits

    iota_e = lax.broadcasted_iota(jnp.int32, (_T, _E), 1)
    m1 = jnp.max(logits, axis=1, keepdims=True)
    i1 = jnp.min(jnp.where(logits == m1, iota_e, _E), axis=1, keepdims=True)
    masked = jnp.where(iota_e == i1, -jnp.inf, logits)
    m2 = jnp.max(masked, axis=1, keepdims=True)
    i2 = jnp.min(jnp.where(masked == m2, iota_e, _E), axis=1, keepdims=True)

    d = jnp.exp(m2 - m1)
    w1 = 1.0 / (1.0 + d)
    w2 = d / (1.0 + d)

    oh1 = (iota_e == i1).astype(jnp.float32)
    oh2 = (iota_e == i2).astype(jnp.float32)
    ohsum_ref[...] = oh1 + oh2

    rr = lax.broadcasted_iota(jnp.int32, (_CBLK, _CBLK), 0)
    cc = lax.broadcasted_iota(jnp.int32, (_CBLK, _CBLK), 1)
    ltri = (rr > cc).astype(jnp.float32)

    def blk(j, carry):
        ohb = ohsum_ref[pl.ds(j * _CBLK, _CBLK), :]
        cum_ref[pl.ds(j * _CBLK, _CBLK), :] = (
            jnp.dot(ltri, ohb, preferred_element_type=jnp.float32) + carry)
        return carry + jnp.sum(ohb, axis=0, keepdims=True)

    lax.fori_loop(0, _T // _CBLK, blk, jnp.zeros((1, _E), jnp.float32))

    cum = cum_ref[...]
    pos1 = jnp.sum(cum * oh1, axis=1, keepdims=True)
    pos2 = jnp.sum(cum * oh2, axis=1, keepdims=True)
    keep1 = pos1 < float(_C)
    keep2 = pos2 < float(_C)
    slot1 = jnp.where(keep1, i1 * _C + pos1.astype(jnp.int32), _DUMMY)
    slot2 = jnp.where(keep2, i2 * _C + pos2.astype(jnp.int32), _DUMMY)
    kw1 = jnp.where(keep1, w1, 0.0)
    kw2 = jnp.where(keep2, w2, 0.0)
    slot1_ref[...] = slot1
    slot2_ref[...] = slot2

    t_row = lax.broadcasted_iota(jnp.int32, (1, _T), 1).astype(jnp.float32)
    a1 = jnp.concatenate([t_row, jnp.reshape(kw1, (1, _T))], axis=0)
    a2 = jnp.concatenate([t_row, jnp.reshape(kw2, (1, _T))], axis=0)
    lane512 = lax.broadcasted_iota(jnp.int32, (_T, 512), 1)
    for g in range(_NSLOT // 512):
        oh1g = (slot1 - g * 512 == lane512).astype(jnp.float32)
        oh2g = (slot2 - g * 512 == lane512).astype(jnp.float32)
        b = (jnp.dot(a1, oh1g, preferred_element_type=jnp.float32,
                     precision=lax.Precision.HIGHEST) +
             jnp.dot(a2, oh2g, preferred_element_type=jnp.float32,
                     precision=lax.Precision.HIGHEST))
        tok2d_ref[pl.ds(g, 1), :] = b[0:1, :].astype(jnp.int32)
        kw2d_ref[pl.ds(g, 1), :] = b[1:2, :]


def _router(x, router_w):
    return pl.pallas_call(
        _router_body,
        out_shape=[
            jax.ShapeDtypeStruct((_T, _E), jnp.float32),
            jax.ShapeDtypeStruct((_T, 1), jnp.int32),
            jax.ShapeDtypeStruct((_T, 1), jnp.int32),
            jax.ShapeDtypeStruct((_NSLOT // 512, 512), jnp.int32),
            jax.ShapeDtypeStruct((_NSLOT // 512, 512), jnp.float32),
        ],
        scratch_shapes=[
            pltpu.VMEM((_T, _E), jnp.float32),
            pltpu.VMEM((_T, _E), jnp.float32),
        ],
    )(x, router_w)


def _expert_body(tok_ref, kw_ref, x_ref, gw_ref, uw_ref, dw_ref, out_ref,
                 xg_ref):
    e = pl.program_id(0)

    @pl.when(e == _E)
    def _zero():
        out_ref[...] = jnp.zeros_like(out_ref)

    @pl.when(e < _E)
    def _compute():
        def gather_row(c, _):
            t = tok_ref[e * _C + c]
            xg_ref[pl.ds(c, 1), :] = x_ref[pl.ds(t, 1), :]
            return 0

        lax.fori_loop(0, _C, gather_row, 0)
        x = xg_ref[...].astype(jnp.bfloat16)
        gw = gw_ref[0].astype(jnp.bfloat16)
        uw = uw_ref[0].astype(jnp.bfloat16)
        dw = dw_ref[0].astype(jnp.bfloat16)
        g = jnp.dot(x, gw, preferred_element_type=jnp.float32)
        u = jnp.dot(x, uw, preferred_element_type=jnp.float32)
        a = g * (1.0 / (1.0 + jnp.exp(-g))) * u
        y = jnp.dot(a.astype(jnp.bfloat16), dw,
                    preferred_element_type=jnp.float32)
        out_ref[...] = y * kw_ref[...]


def _experts(tok_map, kwslot_col, x, gate_w, up_w, down_w):
    last = lambda e: (jnp.minimum(e, _E - 1), 0, 0)
    return pl.pallas_call(
        _expert_body,
        grid_spec=pltpu.PrefetchScalarGridSpec(
            num_scalar_prefetch=1,
            grid=(_E + 1,),
            in_specs=[
                pl.BlockSpec((_C, 1), lambda e, tok: (jnp.minimum(e, _E - 1), 0)),
                pl.BlockSpec((_T, _H), lambda e, tok: (0, 0)),
                pl.BlockSpec((1, _H, _I), lambda e, tok: (jnp.minimum(e, _E - 1), 0, 0)),
                pl.BlockSpec((1, _H, _I), lambda e, tok: (jnp.minimum(e, _E - 1), 0, 0)),
                pl.BlockSpec((1, _I, _H), lambda e, tok: (jnp.minimum(e, _E - 1), 0, 0)),
            ],
            out_specs=pl.BlockSpec((_C, _H), lambda e, tok: (e, 0)),
            scratch_shapes=[pltpu.VMEM((_C, _H), jnp.float32)],
        ),
        out_shape=jax.ShapeDtypeStruct((_NSLOT + _C, _H), jnp.float32),
        compiler_params=pltpu.CompilerParams(
            dimension_semantics=("arbitrary",)),
    )(tok_map, kwslot_col, x, gate_w, up_w, down_w)


def _combine_body(eo, s1, s2, out, idx1_v, idx2_v, r1_v, r2_v, sem1, sem2):
    wid = lax.axis_index("s") * 2 + lax.axis_index("c")
    base = wid * _TOK_PER_W
    pltpu.sync_copy(s1.at[pl.ds(base, _TOK_PER_W)], idx1_v)
    pltpu.sync_copy(s2.at[pl.ds(base, _TOK_PER_W)], idx2_v)
    c1 = pltpu.async_copy(eo.at[idx1_v], r1_v, sem1)
    c2 = pltpu.async_copy(eo.at[idx2_v], r2_v, sem2)
    c1.wait()
    c2.wait()

    def row(t, _):
        for j in range(_H // 16):
            sl = pl.ds(j * 16, 16)
            r1_v[t, sl] = r1_v[t, sl] + r2_v[t, sl]
        return 0

    lax.fori_loop(0, _TOK_PER_W, row, 0)
    pltpu.sync_copy(r1_v, out.at[pl.ds(base, _TOK_PER_W)])


def _combine(eo, slot1, slot2):
    mesh = plsc.VectorSubcoreMesh(core_axis_name="c", subcore_axis_name="s")
    f = pl.kernel(
        _combine_body,
        out_type=jax.ShapeDtypeStruct((_T, _H), jnp.float32),
        mesh=mesh,
        scratch_types=[
            pltpu.VMEM((_TOK_PER_W,), jnp.int32),
            pltpu.VMEM((_TOK_PER_W,), jnp.int32),
            pltpu.VMEM((_TOK_PER_W, _H), jnp.float32),
            pltpu.VMEM((_TOK_PER_W, _H), jnp.float32),
            pltpu.SemaphoreType.DMA,
            pltpu.SemaphoreType.DMA,
        ],
        compiler_params=pltpu.CompilerParams(needs_layout_passes=False),
    )
    return f(eo, slot1, slot2)


def kernel(hidden_states, router_w, gate_w, up_w, down_w):
    S, B, H = hidden_states.shape
    x = hidden_states.reshape(S * B, H)

    logits, slot1c, slot2c, tok2d, kw2d = _router(x, router_w)
    slot1 = slot1c.reshape(_T)
    slot2 = slot2c.reshape(_T)

    eo = _experts(tok2d.reshape(_NSLOT), kw2d.reshape(_NSLOT, 1),
                  x, gate_w, up_w, down_w)

    out = _combine(eo, slot1, slot2)
    return out.reshape(S, B, H), logits


# speedup vs baseline: 1.0283x; 1.0283x over previous
import functools
import math

import jax
import jax.numpy as jnp
from jax import lax
from jax.experimental import pallas as pl
from jax.experimental.pallas import tpu as pltpu
from jax.experimental.pallas import tpu_sc as plsc

_E = 64
_K = 2
_H = 768
_I = 1536
_T = 2048
_C = 64
_NSLOT = _E * _C
_DUMMY = _NSLOT
_NW = 32
_SLOTS_PER_W = _NSLOT // _NW
_TOK_PER_W = _T // _NW
_CBLK = 256


def _router_body(x_ref, rw_ref, logits_ref, slot1_ref, slot2_ref,
                 kw1_ref, kw2_ref, ohsum_ref, cum_ref):
    x = x_ref[...]
    rw = rw_ref[...]
    logits = jnp.dot(x, rw, preferred_element_type=jnp.float32)
    logits_ref[...] = logits

    iota_e = lax.broadcasted_iota(jnp.int32, (_T, _E), 1)
    m1 = jnp.max(logits, axis=1, keepdims=True)
    i1 = jnp.min(jnp.where(logits == m1, iota_e, _E), axis=1, keepdims=True)
    masked = jnp.where(iota_e == i1, -jnp.inf, logits)
    m2 = jnp.max(masked, axis=1, keepdims=True)
    i2 = jnp.min(jnp.where(masked == m2, iota_e, _E), axis=1, keepdims=True)

    d = jnp.exp(m2 - m1)
    w1 = 1.0 / (1.0 + d)
    w2 = d / (1.0 + d)

    oh1 = (iota_e == i1).astype(jnp.float32)
    oh2 = (iota_e == i2).astype(jnp.float32)
    ohsum_ref[...] = oh1 + oh2

    rr = lax.broadcasted_iota(jnp.int32, (_CBLK, _CBLK), 0)
    cc = lax.broadcasted_iota(jnp.int32, (_CBLK, _CBLK), 1)
    ltri = (rr > cc).astype(jnp.float32)

    def blk(j, carry):
        ohb = ohsum_ref[pl.ds(j * _CBLK, _CBLK), :]
        cum_ref[pl.ds(j * _CBLK, _CBLK), :] = (
            jnp.dot(ltri, ohb, preferred_element_type=jnp.float32) + carry)
        return carry + jnp.sum(ohb, axis=0, keepdims=True)

    lax.fori_loop(0, _T // _CBLK, blk, jnp.zeros((1, _E), jnp.float32))

    cum = cum_ref[...]
    pos1 = jnp.sum(cum * oh1, axis=1, keepdims=True)
    pos2 = jnp.sum(cum * oh2, axis=1, keepdims=True)
    keep1 = pos1 < float(_C)
    keep2 = pos2 < float(_C)
    slot1_ref[...] = jnp.where(keep1, i1 * _C + pos1.astype(jnp.int32), _DUMMY)
    slot2_ref[...] = jnp.where(keep2, i2 * _C + pos2.astype(jnp.int32), _DUMMY)
    kw1_ref[...] = jnp.where(keep1, w1, 0.0)
    kw2_ref[...] = jnp.where(keep2, w2, 0.0)


def _router(x, router_w):
    return pl.pallas_call(
        _router_body,
        out_shape=[
            jax.ShapeDtypeStruct((_T, _E), jnp.float32),
            jax.ShapeDtypeStruct((_T, 1), jnp.int32),
            jax.ShapeDtypeStruct((_T, 1), jnp.int32),
            jax.ShapeDtypeStruct((_T, 1), jnp.float32),
            jax.ShapeDtypeStruct((_T, 1), jnp.float32),
        ],
        scratch_shapes=[
            pltpu.VMEM((_T, _E), jnp.float32),
            pltpu.VMEM((_T, _E), jnp.float32),
        ],
    )(x, router_w)


def _dispatch_body(s1, s2, w1, w2, tok_out, kwslot,
                   s1_v, s2_v, w1_v, w2_v, tok_map, kw_map):
    wid = lax.axis_index("s") * 2 + lax.axis_index("c")
    base = wid * _SLOTS_PER_W

    pltpu.sync_copy(s1, s1_v)
    pltpu.sync_copy(s2, s2_v)
    pltpu.sync_copy(w1, w1_v)
    pltpu.sync_copy(w2, w2_v)

    iota16 = lax.broadcasted_iota(jnp.int32, (16,), 0)
    sentinel = jnp.zeros((16,), jnp.int32)
    zero16 = jnp.zeros((16,), jnp.float32)

    def init(j, _):
        tok_map[pl.ds(j * 16, 16)] = sentinel
        kw_map[pl.ds(j * 16, 16)] = zero16
        return 0

    lax.fori_loop(0, _SLOTS_PER_W // 16, init, 0)

    def scan(i, _):
        for u in range(4):
            off = (4 * i + u) * 16
            tok = off + iota16
            for sv, wv in ((s1_v, w1_v), (s2_v, w2_v)):
                s = sv[pl.ds(off, 16)] - base
                m = (s >= 0) & (s < _SLOTS_PER_W)
                plsc.store_scatter(tok_map, [s], tok, mask=m)
                plsc.store_scatter(kw_map, [s], wv[pl.ds(off, 16)], mask=m)
        return 0

    lax.fori_loop(0, _T // 64, scan, 0)

    pltpu.sync_copy(tok_map, tok_out.at[pl.ds(base, _SLOTS_PER_W)])
    pltpu.sync_copy(kw_map, kwslot.at[pl.ds(base, _SLOTS_PER_W)])


def _dispatch(slot1, slot2, kw1, kw2):
    mesh = plsc.VectorSubcoreMesh(core_axis_name="c", subcore_axis_name="s")
    f = pl.kernel(
        _dispatch_body,
        out_type=[
            jax.ShapeDtypeStruct((_NSLOT,), jnp.int32),
            jax.ShapeDtypeStruct((_NSLOT,), jnp.float32),
        ],
        mesh=mesh,
        scratch_types=[
            pltpu.VMEM((_T,), jnp.int32),
            pltpu.VMEM((_T,), jnp.int32),
            pltpu.VMEM((_T,), jnp.float32),
            pltpu.VMEM((_T,), jnp.float32),
            pltpu.VMEM((_SLOTS_PER_W,), jnp.int32),
            pltpu.VMEM((_SLOTS_PER_W,), jnp.float32),
        ],
        compiler_params=pltpu.CompilerParams(needs_layout_passes=False),
    )
    return f(slot1, slot2, kw1, kw2)


def _expert_body(tok_ref, kw_ref, x_ref, gw_ref, uw_ref, dw_ref, out_ref,
                 xg_ref):
    e = pl.program_id(0)

    @pl.when(e == _E)
    def _zero():
        out_ref[...] = jnp.zeros_like(out_ref)

    @pl.when(e < _E)
    def _compute():
        def gather_row(c, _):
            t = tok_ref[e * _C + c]
            xg_ref[pl.ds(c, 1), :] = x_ref[pl.ds(t, 1), :]
            return 0

        lax.fori_loop(0, _C, gather_row, 0)
        x = xg_ref[...].astype(jnp.bfloat16)
        gw = gw_ref[0].astype(jnp.bfloat16)
        uw = uw_ref[0].astype(jnp.bfloat16)
        dw = dw_ref[0].astype(jnp.bfloat16)
        g = jnp.dot(x, gw, preferred_element_type=jnp.float32)
        u = jnp.dot(x, uw, preferred_element_type=jnp.float32)
        a = g * (1.0 / (1.0 + jnp.exp(-g))) * u
        y = jnp.dot(a.astype(jnp.bfloat16), dw,
                    preferred_element_type=jnp.float32)
        out_ref[...] = y * kw_ref[...]


def _experts(tok_map, kwslot_col, x, gate_w, up_w, down_w):
    last = lambda e: (jnp.minimum(e, _E - 1), 0, 0)
    return pl.pallas_call(
        _expert_body,
        grid_spec=pltpu.PrefetchScalarGridSpec(
            num_scalar_prefetch=1,
            grid=(_E + 1,),
            in_specs=[
                pl.BlockSpec((_C, 1), lambda e, tok: (jnp.minimum(e, _E - 1), 0)),
                pl.BlockSpec((_T, _H), lambda e, tok: (0, 0)),
                pl.BlockSpec((1, _H, _I), lambda e, tok: (jnp.minimum(e, _E - 1), 0, 0)),
                pl.BlockSpec((1, _H, _I), lambda e, tok: (jnp.minimum(e, _E - 1), 0, 0)),
                pl.BlockSpec((1, _I, _H), lambda e, tok: (jnp.minimum(e, _E - 1), 0, 0)),
            ],
            out_specs=pl.BlockSpec((_C, _H), lambda e, tok: (e, 0)),
            scratch_shapes=[pltpu.VMEM((_C, _H), jnp.float32)],
        ),
        out_shape=jax.ShapeDtypeStruct((_NSLOT + _C, _H), jnp.float32),
        compiler_params=pltpu.CompilerParams(
            dimension_semantics=("arbitrary",)),
    )(tok_map, kwslot_col, x, gate_w, up_w, down_w)


def _combine_body(eo, s1, s2, out, idx1_v, idx2_v, r1_v, r2_v, sem1, sem2):
    wid = lax.axis_index("s") * 2 + lax.axis_index("c")
    base = wid * _TOK_PER_W
    pltpu.sync_copy(s1.at[pl.ds(base, _TOK_PER_W)], idx1_v)
    pltpu.sync_copy(s2.at[pl.ds(base, _TOK_PER_W)], idx2_v)
    c1 = pltpu.async_copy(eo.at[idx1_v], r1_v, sem1)
    c2 = pltpu.async_copy(eo.at[idx2_v], r2_v, sem2)
    c1.wait()
    c2.wait()

    def row(t, _):
        for j in range(_H // 16):
            sl = pl.ds(j * 16, 16)
            r1_v[t, sl] = r1_v[t, sl] + r2_v[t, sl]
        return 0

    lax.fori_loop(0, _TOK_PER_W, row, 0)
    pltpu.sync_copy(r1_v, out.at[pl.ds(base, _TOK_PER_W)])


def _combine(eo, slot1, slot2):
    mesh = plsc.VectorSubcoreMesh(core_axis_name="c", subcore_axis_name="s")
    f = pl.kernel(
        _combine_body,
        out_type=jax.ShapeDtypeStruct((_T, _H), jnp.float32),
        mesh=mesh,
        scratch_types=[
            pltpu.VMEM((_TOK_PER_W,), jnp.int32),
            pltpu.VMEM((_TOK_PER_W,), jnp.int32),
            pltpu.VMEM((_TOK_PER_W, _H), jnp.float32),
            pltpu.VMEM((_TOK_PER_W, _H), jnp.float32),
            pltpu.SemaphoreType.DMA,
            pltpu.SemaphoreType.DMA,
        ],
        compiler_params=pltpu.CompilerParams(needs_layout_passes=False,
                                             use_tc_tiling_on_sc=True),
    )
    return f(eo, slot1, slot2)


def kernel(hidden_states, router_w, gate_w, up_w, down_w):
    S, B, H = hidden_states.shape
    x = hidden_states.reshape(S * B, H)

    logits, slot1c, slot2c, kw1c, kw2c = _router(x, router_w)
    slot1 = slot1c.reshape(_T)
    slot2 = slot2c.reshape(_T)

    tok_map, kwslot = _dispatch(slot1, slot2,
                                kw1c.reshape(_T), kw2c.reshape(_T))
    eo = _experts(tok_map, kwslot.reshape(_NSLOT, 1), x, gate_w, up_w, down_w)

    out = _combine(eo, slot1, slot2)
    return out.reshape(S, B, H), logits


# speedup vs baseline: 1.0787x; 1.0490x over previous
import functools
import math

import jax
import jax.numpy as jnp
from jax import lax
from jax.experimental import pallas as pl
from jax.experimental.pallas import tpu as pltpu
from jax.experimental.pallas import tpu_sc as plsc

_E = 64
_K = 2
_H = 768
_I = 1536
_T = 2048
_C = 64
_NSLOT = _E * _C
_DUMMY = _NSLOT
_NW = 32
_SLOTS_PER_W = _NSLOT // _NW
_TOK_PER_W = _T // _NW
_CBLK = 256


def _router_body(x_ref, rw_ref, logits_ref, slot1_ref, slot2_ref,
                 tok2d_ref, kw2d_ref, ohsum_ref, cum_ref):
    x = x_ref[...]
    rw = rw_ref[...]
    logits = jnp.dot(x, rw, preferred_element_type=jnp.float32)
    logits_ref[...] = logits

    iota_e = lax.broadcasted_iota(jnp.int32, (_T, _E), 1)
    m1 = jnp.max(logits, axis=1, keepdims=True)
    i1 = jnp.min(jnp.where(logits == m1, iota_e, _E), axis=1, keepdims=True)
    masked = jnp.where(iota_e == i1, -jnp.inf, logits)
    m2 = jnp.max(masked, axis=1, keepdims=True)
    i2 = jnp.min(jnp.where(masked == m2, iota_e, _E), axis=1, keepdims=True)

    d = jnp.exp(m2 - m1)
    w1 = 1.0 / (1.0 + d)
    w2 = d / (1.0 + d)

    oh1 = (iota_e == i1).astype(jnp.float32)
    oh2 = (iota_e == i2).astype(jnp.float32)
    ohsum_ref[...] = oh1 + oh2

    rr = lax.broadcasted_iota(jnp.int32, (_CBLK, _CBLK), 0)
    cc = lax.broadcasted_iota(jnp.int32, (_CBLK, _CBLK), 1)
    ltri = (rr > cc).astype(jnp.float32)

    def blk(j, carry):
        ohb = ohsum_ref[pl.ds(j * _CBLK, _CBLK), :]
        cum_ref[pl.ds(j * _CBLK, _CBLK), :] = (
            jnp.dot(ltri, ohb, preferred_element_type=jnp.float32) + carry)
        return carry + jnp.sum(ohb, axis=0, keepdims=True)

    lax.fori_loop(0, _T // _CBLK, blk, jnp.zeros((1, _E), jnp.float32))

    cum = cum_ref[...]
    pos1 = jnp.sum(cum * oh1, axis=1, keepdims=True)
    pos2 = jnp.sum(cum * oh2, axis=1, keepdims=True)
    keep1 = pos1 < float(_C)
    keep2 = pos2 < float(_C)
    slot1 = jnp.where(keep1, i1 * _C + pos1.astype(jnp.int32), _DUMMY)
    slot2 = jnp.where(keep2, i2 * _C + pos2.astype(jnp.int32), _DUMMY)
    kw1 = jnp.where(keep1, w1, 0.0)
    kw2 = jnp.where(keep2, w2, 0.0)
    slot1_ref[...] = slot1
    slot2_ref[...] = slot2

    t_row = lax.broadcasted_iota(jnp.int32, (1, _T), 1)
    t_hi = (t_row // 64).astype(jnp.float32)
    t_lo = (t_row % 64).astype(jnp.float32)

    def split_rows(kw):
        w_row = jnp.reshape(kw, (1, _T))
        w_hi = w_row.astype(jnp.bfloat16).astype(jnp.float32)
        return jnp.concatenate([t_hi, t_lo, w_hi, w_row - w_hi], axis=0)

    a1 = split_rows(kw1).astype(jnp.bfloat16)
    a2 = split_rows(kw2).astype(jnp.bfloat16)
    lane512 = lax.broadcasted_iota(jnp.int32, (_T, 512), 1)
    for g in range(_NSLOT // 512):
        oh1g = (slot1 - g * 512 == lane512).astype(jnp.bfloat16)
        oh2g = (slot2 - g * 512 == lane512).astype(jnp.bfloat16)
        b = (jnp.dot(a1, oh1g, preferred_element_type=jnp.float32) +
             jnp.dot(a2, oh2g, preferred_element_type=jnp.float32))
        tok = b[0:1, :] * 64.0 + b[1:2, :]
        tok2d_ref[pl.ds(g, 1), :] = tok.astype(jnp.int32)
        kw2d_ref[pl.ds(g, 1), :] = b[2:3, :] + b[3:4, :]


def _router(x, router_w):
    return pl.pallas_call(
        _router_body,
        out_shape=[
            jax.ShapeDtypeStruct((_T, _E), jnp.float32),
            jax.ShapeDtypeStruct((_T, 1), jnp.int32),
            jax.ShapeDtypeStruct((_T, 1), jnp.int32),
            jax.ShapeDtypeStruct((_NSLOT // 512, 512), jnp.int32),
            jax.ShapeDtypeStruct((_NSLOT // 512, 512), jnp.float32),
        ],
        scratch_shapes=[
            pltpu.VMEM((_T, _E), jnp.float32),
            pltpu.VMEM((_T, _E), jnp.float32),
        ],
    )(x, router_w)


def _expert_body(tok_ref, kw_ref, x_ref, gw_ref, uw_ref, dw_ref, out_ref,
                 xg_ref):
    e = pl.program_id(0)

    @pl.when(e == _E)
    def _zero():
        out_ref[...] = jnp.zeros_like(out_ref)

    @pl.when(e < _E)
    def _compute():
        def gather_row(c, _):
            t = tok_ref[e * _C + c]
            xg_ref[pl.ds(c, 1), :] = x_ref[pl.ds(t, 1), :]
            return 0

        lax.fori_loop(0, _C, gather_row, 0)
        x = xg_ref[...].astype(jnp.bfloat16)
        gw = gw_ref[0].astype(jnp.bfloat16)
        uw = uw_ref[0].astype(jnp.bfloat16)
        dw = dw_ref[0].astype(jnp.bfloat16)
        g = jnp.dot(x, gw, preferred_element_type=jnp.float32)
        u = jnp.dot(x, uw, preferred_element_type=jnp.float32)
        a = g * (1.0 / (1.0 + jnp.exp(-g))) * u
        y = jnp.dot(a.astype(jnp.bfloat16), dw,
                    preferred_element_type=jnp.float32)
        out_ref[...] = y * kw_ref[...]


def _experts(tok_map, kwslot_col, x, gate_w, up_w, down_w):
    last = lambda e: (jnp.minimum(e, _E - 1), 0, 0)
    return pl.pallas_call(
        _expert_body,
        grid_spec=pltpu.PrefetchScalarGridSpec(
            num_scalar_prefetch=1,
            grid=(_E + 1,),
            in_specs=[
                pl.BlockSpec((_C, 1), lambda e, tok: (jnp.minimum(e, _E - 1), 0)),
                pl.BlockSpec((_T, _H), lambda e, tok: (0, 0)),
                pl.BlockSpec((1, _H, _I), lambda e, tok: (jnp.minimum(e, _E - 1), 0, 0)),
                pl.BlockSpec((1, _H, _I), lambda e, tok: (jnp.minimum(e, _E - 1), 0, 0)),
                pl.BlockSpec((1, _I, _H), lambda e, tok: (jnp.minimum(e, _E - 1), 0, 0)),
            ],
            out_specs=pl.BlockSpec((_C, _H), lambda e, tok: (e, 0)),
            scratch_shapes=[pltpu.VMEM((_C, _H), jnp.float32)],
        ),
        out_shape=jax.ShapeDtypeStruct((_NSLOT + _C, _H), jnp.float32),
        compiler_params=pltpu.CompilerParams(
            dimension_semantics=("arbitrary",)),
    )(tok_map, kwslot_col, x, gate_w, up_w, down_w)


def _combine_body(eo, s1, s2, out, idx1_v, idx2_v, r1_v, r2_v, sem1, sem2):
    wid = lax.axis_index("s") * 2 + lax.axis_index("c")
    base = wid * _TOK_PER_W
    pltpu.sync_copy(s1.at[pl.ds(base, _TOK_PER_W)], idx1_v)
    pltpu.sync_copy(s2.at[pl.ds(base, _TOK_PER_W)], idx2_v)
    c1 = pltpu.async_copy(eo.at[idx1_v], r1_v, sem1)
    c2 = pltpu.async_copy(eo.at[idx2_v], r2_v, sem2)
    c1.wait()
    c2.wait()

    def row(t, _):
        for j in range(_H // 16):
            sl = pl.ds(j * 16, 16)
            r1_v[t, sl] = r1_v[t, sl] + r2_v[t, sl]
        return 0

    lax.fori_loop(0, _TOK_PER_W, row, 0)
    pltpu.sync_copy(r1_v, out.at[pl.ds(base, _TOK_PER_W)])


def _combine(eo, slot1, slot2):
    mesh = plsc.VectorSubcoreMesh(core_axis_name="c", subcore_axis_name="s")
    f = pl.kernel(
        _combine_body,
        out_type=jax.ShapeDtypeStruct((_T, _H), jnp.float32),
        mesh=mesh,
        scratch_types=[
            pltpu.VMEM((_TOK_PER_W,), jnp.int32),
            pltpu.VMEM((_TOK_PER_W,), jnp.int32),
            pltpu.VMEM((_TOK_PER_W, _H), jnp.float32),
            pltpu.VMEM((_TOK_PER_W, _H), jnp.float32),
            pltpu.SemaphoreType.DMA,
            pltpu.SemaphoreType.DMA,
        ],
        compiler_params=pltpu.CompilerParams(needs_layout_passes=False,
                                             use_tc_tiling_on_sc=True),
    )
    return f(eo, slot1, slot2)


def kernel(hidden_states, router_w, gate_w, up_w, down_w):
    S, B, H = hidden_states.shape
    x = hidden_states.reshape(S * B, H)

    logits, slot1c, slot2c, tok2d, kw2d = _router(x, router_w)
    slot1 = slot1c.reshape(_T)
    slot2 = slot2c.reshape(_T)

    eo = _experts(tok2d.reshape(_NSLOT), kw2d.reshape(_NSLOT, 1),
                  x, gate_w, up_w, down_w)

    out = _combine(eo, slot1, slot2)
    return out.reshape(S, B, H), logits
